# Initial kernel scaffold; baseline (speedup 1.0000x reference)
import jax
import jax.numpy as jnp
from jax.experimental import pallas as pl


def kernel(x, table, W1, b1, W2, b2):
    raise NotImplementedError("write your pallas kernel here")


# speedup vs baseline: 1.9413x; 1.9413x over previous
import functools

import jax
import jax.numpy as jnp
from jax import lax
from jax.experimental import pallas as pl
from jax.experimental.pallas import tpu as pltpu
from jax.experimental.pallas import tpu_sc as plsc

_VOCAB = 1000000
_EMBED = 64
_B = 4096
_L = 50
_WIN = 3
_HID = 1024
_NCLS = 1000

_NIDX = _B * _L
_GW = 128
_BB = 256
_FLATW = (_L + _WIN - 1) * _EMBED
_PADW = (_L + 2 * (_WIN - 1)) * _EMBED


def _sc_gather(table, idx):
    mesh = plsc.VectorSubcoreMesh(core_axis_name="c", subcore_axis_name="s")

    @functools.partial(
        pl.kernel,
        out_type=jax.ShapeDtypeStruct((_NIDX, _EMBED), jnp.float32),
        mesh=mesh,
        compiler_params=pltpu.CompilerParams(use_tc_tiling_on_sc=False),
    )
    def gather_kernel(table_hbm, idx_hbm, out_hbm):
        def body(i_vmem, o_vmem):
            pltpu.sync_copy(table_hbm.at[i_vmem.at[0]], o_vmem)

        pltpu.emit_pipeline(
            body,
            grid=(_NIDX // _GW,),
            in_specs=[pl.BlockSpec((1, _GW), index_map=lambda i: (0, i))],
            out_specs=[pl.BlockSpec((_GW, _EMBED), index_map=lambda i: (i, 0))],
            core_axis_name=("c", "s"),
            dimension_semantics=(pltpu.PARALLEL,),
        )(idx_hbm, out_hbm)

    return gather_kernel(table, idx)


def _mlp_body(emb_ref, r0_ref, w1_ref, b1_ref, w2_ref, b2_ref, out_ref, p_ref):
    r0 = jnp.broadcast_to(r0_ref[...], (_BB, _EMBED))
    p_ref[:, : _EMBED] = r0
    p_ref[:, _EMBED : 2 * _EMBED] = r0
    p_ref[:, 2 * _EMBED : 2 * _EMBED + _L * _EMBED] = emb_ref[...]
    p_ref[:, _PADW - 2 * _EMBED : _PADW - _EMBED] = r0
    p_ref[:, _PADW - _EMBED :] = r0
    p = p_ref[...]
    flat = jnp.maximum(
        jnp.maximum(p[:, :_FLATW], p[:, _EMBED : _EMBED + _FLATW]),
        p[:, 2 * _EMBED : 2 * _EMBED + _FLATW],
    )
    h = jnp.dot(
        flat.astype(jnp.bfloat16), w1_ref[...], preferred_element_type=jnp.float32
    ) + b1_ref[...]
    h = jnp.maximum(h, 0.0).astype(jnp.bfloat16)
    out_ref[...] = jnp.dot(
        h, w2_ref[...], preferred_element_type=jnp.float32
    ) + b2_ref[...]


def _tc_mlp(emb2d, row0, w1, b1, w2, b2):
    grid = (_B // _BB,)
    return pl.pallas_call(
        _mlp_body,
        grid=grid,
        in_specs=[
            pl.BlockSpec((_BB, _L * _EMBED), lambda i: (i, 0)),
            pl.BlockSpec((1, _EMBED), lambda i: (0, 0)),
            pl.BlockSpec((_FLATW, _HID), lambda i: (0, 0)),
            pl.BlockSpec((1, _HID), lambda i: (0, 0)),
            pl.BlockSpec((_HID, _NCLS), lambda i: (0, 0)),
            pl.BlockSpec((1, _NCLS), lambda i: (0, 0)),
        ],
        out_specs=pl.BlockSpec((_BB, _NCLS), lambda i: (i, 0)),
        out_shape=jax.ShapeDtypeStruct((_B, _NCLS), jnp.float32),
        scratch_shapes=[pltpu.VMEM((_BB, _PADW), jnp.float32)],
    )(emb2d, row0, w1, b1, w2, b2)


def kernel(x, table, W1, b1, W2, b2):
    idx = x.astype(jnp.int32).reshape(1, _NIDX)
    emb = _sc_gather(table, idx)
    emb2d = emb.reshape(_B, _L * _EMBED)
    row0 = lax.slice(table, (0, 0), (1, _EMBED))
    w1 = W1.astype(jnp.bfloat16)
    w2 = W2.astype(jnp.bfloat16)
    return _tc_mlp(
        emb2d, row0, w1, b1.reshape(1, _HID), w2, b2.reshape(1, _NCLS)
    )
